# baseline (speedup 1.0000x reference)
import functools

import jax
import jax.numpy as jnp
from jax import lax
from jax.experimental import pallas as pl
from jax.experimental.pallas import tpu as pltpu
from jax.experimental.pallas import tpu_sc as plsc

N = 10000
E = 320000
D = 128
DOUT = 64
NC = 2
NS = 16
NW = NC * NS
NPAD = 10240
RPT = NPAD // NS
EPT = E // NW
K = 40

def _mesh():
    return plsc.VectorSubcoreMesh(core_axis_name="c", subcore_axis_name="s",
                                  num_cores=NC, num_subcores=NS)


CH = 2000


@functools.cache
def _make_sc_degrees():
    return functools.partial(
        pl.kernel,
        out_type=jax.ShapeDtypeStruct((2, NW, NPAD), jnp.float32),
        mesh=_mesh(),
        compiler_params=pltpu.CompilerParams(needs_layout_passes=False,
                                             use_tc_tiling_on_sc=False),
        scratch_types=[
            pltpu.VMEM((NPAD,), jnp.float32),
            pltpu.VMEM((NPAD,), jnp.float32),
            pltpu.VMEM((CH,), jnp.int32),
            pltpu.VMEM((CH,), jnp.int32),
        ],
    )(_sc_degrees_body)


def _sc_degrees_body(src_hbm, dst_hbm, out_hbm, cnt_s, cnt_d, ebuf_s, ebuf_d):
    c = lax.axis_index("c")
    s = lax.axis_index("s")
    wid = c * NS + s
    base = wid * EPT
    zeros16 = jnp.zeros((16,), jnp.float32)
    ones16 = jnp.ones((16,), jnp.float32)

    def _z(i, _):
        cnt_s[pl.ds(i * 16, 16)] = zeros16
        cnt_d[pl.ds(i * 16, 16)] = zeros16
        return _
    lax.fori_loop(0, NPAD // 16, _z, None)

    def _outer(oi, _):
        pltpu.sync_copy(src_hbm.at[pl.ds(base + oi * CH, CH)], ebuf_s)
        pltpu.sync_copy(dst_hbm.at[pl.ds(base + oi * CH, CH)], ebuf_d)

        def _inner(j, _):
            vs = ebuf_s[pl.ds(j * 16, 16)]
            vd = ebuf_d[pl.ds(j * 16, 16)]
            plsc.addupdate_scatter(cnt_s, [vs], ones16)
            plsc.addupdate_scatter(cnt_d, [vd], ones16)
            return _
        lax.fori_loop(0, CH // 16, _inner, None)
        return _
    lax.fori_loop(0, EPT // CH, _outer, None)

    pltpu.sync_copy(cnt_s, out_hbm.at[0, wid])
    pltpu.sync_copy(cnt_d, out_hbm.at[1, wid])


NCHUNK = EPT // K
NB = 5


@functools.cache
def _make_sc_agg(src_rows):
    @functools.partial(
        pl.kernel,
        out_type=jax.ShapeDtypeStruct((NC, NPAD, D), jnp.float32),
        mesh=_mesh(),
        compiler_params=pltpu.CompilerParams(use_tc_tiling_on_sc=False),
        scratch_types=[
            pltpu.VMEM_SHARED((NPAD, D), jnp.float32),
            pltpu.VMEM((NB, K), jnp.int32),
            pltpu.VMEM((NB, K), jnp.int32),
        ] + [pltpu.VMEM((K, D), jnp.float32)] * NB
          + [pltpu.SemaphoreType.DMA] * (4 * NB),
    )
    def _agg(h_hbm, src_hbm, dst_hbm, out_hbm, acc, sidx, didx, *bufs):
        rows = bufs[:NB]
        sg = bufs[NB:2 * NB]
        ss = bufs[2 * NB:3 * NB]
        sd = bufs[3 * NB:4 * NB]
        sr = bufs[4 * NB:5 * NB]
        c = lax.axis_index("c")
        s = lax.axis_index("s")
        wid = c * NS + s

        zeros16 = jnp.zeros((16,), jnp.float32)

        def _zrow(r, _):
            for cc in range(D // 16):
                rows[0][r, pl.ds(cc * 16, 16)] = zeros16
            return _
        lax.fori_loop(0, K, _zrow, None)

        def _zacc(j, _):
            pltpu.sync_copy(rows[0], acc.at[pl.ds(s * RPT + j * K, K)])
            return _
        lax.fori_loop(0, RPT // K, _zacc, None)
        plsc.subcore_barrier()

        def _wait(sem, like):
            pltpu.make_async_copy(h_hbm.at[pl.ds(0, K)], like, sem).wait()

        def _wait_idx(sem, like):
            pltpu.make_async_copy(dst_hbm.at[wid, 0], like, sem).wait()

        for j in range(NB):
            pltpu.sync_copy(src_hbm.at[wid, j], sidx.at[j])
            pltpu.async_copy(dst_hbm.at[wid, j], didx.at[j], sd[j])
            pltpu.async_copy(h_hbm.at[sidx.at[j]], rows[j], sg[j])

        def _body(t, _):
            for j in range(NB):
                i = NB * t + j

                @pl.when(i < NCHUNK)
                def _():
                    _wait(sg[j], rows[j])

                    @pl.when(i + NB < NCHUNK)
                    def _():
                        pltpu.async_copy(src_hbm.at[wid, i + NB],
                                         sidx.at[j], sr[j])
                    _wait_idx(sd[j], didx.at[j])
                    pltpu.async_copy(rows[j], acc.at[didx.at[j]], ss[j],
                                     add=True)

                    @pl.when(i + NB < NCHUNK)
                    def _():
                        _wait(ss[j], rows[j])
                        pltpu.async_copy(dst_hbm.at[wid, i + NB],
                                         didx.at[j], sd[j])
                        _wait_idx(sr[j], sidx.at[j])
                        pltpu.async_copy(h_hbm.at[sidx.at[j]],
                                         rows[j], sg[j])
            return _
        lax.fori_loop(0, (NCHUNK + NB - 1) // NB, _body, None)
        for j in range(NB):
            _wait(ss[j], rows[j])
        plsc.subcore_barrier()

        def _out(j, _):
            r0 = s * RPT + j * K
            pltpu.sync_copy(acc.at[pl.ds(r0, K)], rows[0])
            pltpu.sync_copy(rows[0], out_hbm.at[c, pl.ds(r0, K)])
            return _
        lax.fori_loop(0, RPT // K, _out, None)

    return _agg


def _tc_prep_body(deg_ref, x_ref, hs_ref, norms_ref):
    ones_w = jnp.ones((NW, 1), jnp.float32)
    cdims = (((0,), (0,)), ((), ()))
    ds_col = lax.dot_general(deg_ref[0], ones_w, cdims,
                             preferred_element_type=jnp.float32)
    dd_col = lax.dot_general(deg_ref[1], ones_w, cdims,
                             preferred_element_type=jnp.float32)
    ns_col = lax.rsqrt(jnp.maximum(ds_col, 1.0))
    nd_col = lax.rsqrt(jnp.maximum(dd_col, 1.0))
    norms_ref[...] = jnp.concatenate([ns_col, nd_col], axis=1)
    hs_ref[...] = x_ref[...] * ns_col[:N]


def _tc_prep(deg_p, x):
    return pl.pallas_call(
        _tc_prep_body,
        out_shape=(jax.ShapeDtypeStruct((N, D), jnp.float32),
                   jax.ShapeDtypeStruct((NPAD, 2), jnp.float32)),
    )(deg_p, x)


def _tc_layer1_body(p_ref, norms_ref, wa_ref, wb_ref, wc_ref,
                    ba_ref, bb_ref, bc_ref, w_ref, g_ref, be_ref, out_ref):
    w = w_ref[...]
    wn = w / jnp.sum(w)
    W = (wn[:, 0:1] * wa_ref[...] + wn[:, 1:2] * wb_ref[...]
         + wn[:, 2:3] * wc_ref[...])
    b = (wn[:, 0:1] * ba_ref[...] + wn[:, 1:2] * bb_ref[...]
         + wn[:, 2:3] * bc_ref[...])
    agg = p_ref[0] + p_ref[1]
    a = agg * norms_ref[:, 1:2]
    h = lax.dot_general(a, W, (((1,), (0,)), ((), ())),
                        preferred_element_type=jnp.float32) + b
    h = jnp.maximum(g_ref[...] * h + be_ref[...], 0.0)
    out_ref[...] = h * norms_ref[:, 0:1]


def _tc_layer1(p, norms, Wa, Wb, Wc, ba, bb, bc, w, gamma, beta):
    return pl.pallas_call(
        _tc_layer1_body,
        out_shape=jax.ShapeDtypeStruct((NPAD, D), jnp.float32),
    )(p, norms, Wa, Wb, Wc, ba.reshape(1, D), bb.reshape(1, D),
      bc.reshape(1, D), w.reshape(1, 3), gamma.reshape(1, D),
      beta.reshape(1, D))


def _tc_layer2_body(p_ref, norms_ref, wa_ref, wb_ref, wc_ref,
                    ba_ref, bb_ref, bc_ref, w_ref, out_ref):
    w = w_ref[...]
    wn = w / jnp.sum(w)
    W = (wn[:, 0:1] * wa_ref[...] + wn[:, 1:2] * wb_ref[...]
         + wn[:, 2:3] * wc_ref[...])
    b = (wn[:, 0:1] * ba_ref[...] + wn[:, 1:2] * bb_ref[...]
         + wn[:, 2:3] * bc_ref[...])
    agg = p_ref[0] + p_ref[1]
    a = agg * norms_ref[:, 1:2]
    o = lax.dot_general(a, W, (((1,), (0,)), ((), ())),
                        preferred_element_type=jnp.float32) + b
    m = jnp.max(o, axis=1, keepdims=True)
    ex = jnp.exp(o - m)
    out_ref[...] = (o - m) - jnp.log(jnp.sum(ex, axis=1, keepdims=True))


def _tc_layer2(p, norms, Wa, Wb, Wc, ba, bb, bc, w):
    return pl.pallas_call(
        _tc_layer2_body,
        out_shape=jax.ShapeDtypeStruct((NPAD, DOUT), jnp.float32),
    )(p, norms, Wa, Wb, Wc, ba.reshape(1, DOUT), bb.reshape(1, DOUT),
      bc.reshape(1, DOUT), w.reshape(1, 3))


def kernel(x, edge_index, W1a, b1a, W1b, b1b, W1c, b1c, w1, bn_gamma,
           bn_beta, W2a, b2a, W2b, b2b, W2c, b2c, w2):
    src = edge_index[0]
    dst = edge_index[1]
    src3 = src.reshape(NW, NCHUNK, K)
    dst3 = dst.reshape(NW, NCHUNK, K)
    deg_p = _make_sc_degrees()(src, dst)
    hs, norms = _tc_prep(deg_p, x)
    p1 = _make_sc_agg(N)(hs, src3, dst3)
    hs2 = _tc_layer1(p1, norms, W1a, W1b, W1c, b1a, b1b, b1c, w1,
                     bn_gamma, bn_beta)
    p2 = _make_sc_agg(NPAD)(hs2, src3, dst3)
    out = _tc_layer2(p2, norms, W2a, W2b, W2c, b2a, b2b, b2c, w2)
    return out[:N]


# speedup vs baseline: 1.2601x; 1.2601x over previous
import functools

import jax
import jax.numpy as jnp
from jax import lax
from jax.experimental import pallas as pl
from jax.experimental.pallas import tpu as pltpu
from jax.experimental.pallas import tpu_sc as plsc

N = 10000
E = 320000
D = 128
DOUT = 64
NC = 2
NS = 16
NW = NC * NS
NPAD = 10240
RPT = NPAD // NS
EPT = E // NW
K = 40

def _mesh():
    return plsc.VectorSubcoreMesh(core_axis_name="c", subcore_axis_name="s",
                                  num_cores=NC, num_subcores=NS)


CH = 2000


@functools.cache
def _make_sc_degrees():
    return functools.partial(
        pl.kernel,
        out_type=jax.ShapeDtypeStruct((2, NW, NPAD), jnp.float32),
        mesh=_mesh(),
        compiler_params=pltpu.CompilerParams(needs_layout_passes=False,
                                             use_tc_tiling_on_sc=False),
        scratch_types=[
            pltpu.VMEM((NPAD,), jnp.float32),
            pltpu.VMEM((NPAD,), jnp.float32),
            pltpu.VMEM((CH,), jnp.int32),
            pltpu.VMEM((CH,), jnp.int32),
        ],
    )(_sc_degrees_body)


def _sc_degrees_body(src_hbm, dst_hbm, out_hbm, cnt_s, cnt_d, ebuf_s, ebuf_d):
    c = lax.axis_index("c")
    s = lax.axis_index("s")
    wid = c * NS + s
    base = wid * EPT
    zeros16 = jnp.zeros((16,), jnp.float32)
    ones16 = jnp.ones((16,), jnp.float32)

    def _z(i, _):
        cnt_s[pl.ds(i * 16, 16)] = zeros16
        cnt_d[pl.ds(i * 16, 16)] = zeros16
        return _
    lax.fori_loop(0, NPAD // 16, _z, None)

    def _outer(oi, _):
        pltpu.sync_copy(src_hbm.at[pl.ds(base + oi * CH, CH)], ebuf_s)
        pltpu.sync_copy(dst_hbm.at[pl.ds(base + oi * CH, CH)], ebuf_d)

        def _inner(j, _):
            vs = ebuf_s[pl.ds(j * 16, 16)]
            vd = ebuf_d[pl.ds(j * 16, 16)]
            plsc.addupdate_scatter(cnt_s, [vs], ones16)
            plsc.addupdate_scatter(cnt_d, [vd], ones16)
            return _
        lax.fori_loop(0, CH // 16, _inner, None)
        return _
    lax.fori_loop(0, EPT // CH, _outer, None)

    pltpu.sync_copy(cnt_s, out_hbm.at[0, wid])
    pltpu.sync_copy(cnt_d, out_hbm.at[1, wid])


NCHUNK = EPT // K
NB = 5


@functools.cache
def _make_sc_agg(src_rows):
    @functools.partial(
        pl.kernel,
        out_type=jax.ShapeDtypeStruct((NC, NPAD, D), jnp.float32),
        mesh=_mesh(),
        compiler_params=pltpu.CompilerParams(use_tc_tiling_on_sc=False),
        scratch_types=[
            pltpu.VMEM_SHARED((NPAD, D), jnp.float32),
            pltpu.VMEM((NCHUNK, K), jnp.int32),
            pltpu.VMEM((NB, K), jnp.int32),
        ] + [pltpu.VMEM((K, D), jnp.float32)] * NB
          + [pltpu.SemaphoreType.DMA] * (3 * NB),
    )
    def _agg(h_hbm, src_hbm, dst_hbm, out_hbm, acc, sidx, didx, *bufs):
        rows = bufs[:NB]
        sg = bufs[NB:2 * NB]
        ss = bufs[2 * NB:3 * NB]
        sd = bufs[3 * NB:4 * NB]
        c = lax.axis_index("c")
        s = lax.axis_index("s")
        wid = c * NS + s

        zeros16 = jnp.zeros((16,), jnp.float32)

        def _zrow(r, _):
            for cc in range(D // 16):
                rows[0][r, pl.ds(cc * 16, 16)] = zeros16
            return _
        lax.fori_loop(0, K, _zrow, None)

        def _zacc(j, _):
            pltpu.sync_copy(rows[0], acc.at[pl.ds(s * RPT + j * K, K)])
            return _
        lax.fori_loop(0, RPT // K, _zacc, None)

        pltpu.sync_copy(src_hbm.at[wid], sidx)
        plsc.subcore_barrier()

        def _wait(sem, like):
            pltpu.make_async_copy(h_hbm.at[pl.ds(0, K)], like, sem).wait()

        def _wait_idx(sem, like):
            pltpu.make_async_copy(dst_hbm.at[wid, 0], like, sem).wait()

        for j in range(NB):
            pltpu.async_copy(dst_hbm.at[wid, j], didx.at[j], sd[j])
            pltpu.async_copy(h_hbm.at[sidx.at[j]], rows[j], sg[j])

        def _body(t, _):
            for j in range(NB):
                i = NB * t + j

                @pl.when(i < NCHUNK)
                def _():
                    _wait(sg[j], rows[j])
                    _wait_idx(sd[j], didx.at[j])
                    pltpu.async_copy(rows[j], acc.at[didx.at[j]], ss[j],
                                     add=True)

                    @pl.when(i + NB < NCHUNK)
                    def _():
                        _wait(ss[j], rows[j])
                        pltpu.async_copy(dst_hbm.at[wid, i + NB],
                                         didx.at[j], sd[j])
                        pltpu.async_copy(h_hbm.at[sidx.at[i + NB]],
                                         rows[j], sg[j])
            return _
        lax.fori_loop(0, (NCHUNK + NB - 1) // NB, _body, None)
        for j in range(NB):
            _wait(ss[j], rows[j])
        plsc.subcore_barrier()

        def _out(j, _):
            r0 = s * RPT + j * K
            pltpu.sync_copy(acc.at[pl.ds(r0, K)], rows[0])
            pltpu.sync_copy(rows[0], out_hbm.at[c, pl.ds(r0, K)])
            return _
        lax.fori_loop(0, RPT // K, _out, None)

    return _agg


def _tc_prep_body(deg_ref, x_ref, hs_ref, norms_ref):
    ones_w = jnp.ones((NW, 1), jnp.float32)
    cdims = (((0,), (0,)), ((), ()))
    ds_col = lax.dot_general(deg_ref[0], ones_w, cdims,
                             preferred_element_type=jnp.float32)
    dd_col = lax.dot_general(deg_ref[1], ones_w, cdims,
                             preferred_element_type=jnp.float32)
    ns_col = lax.rsqrt(jnp.maximum(ds_col, 1.0))
    nd_col = lax.rsqrt(jnp.maximum(dd_col, 1.0))
    norms_ref[...] = jnp.concatenate([ns_col, nd_col], axis=1)
    hs_ref[...] = x_ref[...] * ns_col[:N]


def _tc_prep(deg_p, x):
    return pl.pallas_call(
        _tc_prep_body,
        out_shape=(jax.ShapeDtypeStruct((N, D), jnp.float32),
                   jax.ShapeDtypeStruct((NPAD, 2), jnp.float32)),
    )(deg_p, x)


def _tc_layer1_body(p_ref, norms_ref, wa_ref, wb_ref, wc_ref,
                    ba_ref, bb_ref, bc_ref, w_ref, g_ref, be_ref, out_ref):
    w = w_ref[...]
    wn = w / jnp.sum(w)
    W = (wn[:, 0:1] * wa_ref[...] + wn[:, 1:2] * wb_ref[...]
         + wn[:, 2:3] * wc_ref[...])
    b = (wn[:, 0:1] * ba_ref[...] + wn[:, 1:2] * bb_ref[...]
         + wn[:, 2:3] * bc_ref[...])
    agg = p_ref[0] + p_ref[1]
    a = agg * norms_ref[:, 1:2]
    h = lax.dot_general(a, W, (((1,), (0,)), ((), ())),
                        preferred_element_type=jnp.float32) + b
    h = jnp.maximum(g_ref[...] * h + be_ref[...], 0.0)
    out_ref[...] = h * norms_ref[:, 0:1]


def _tc_layer1(p, norms, Wa, Wb, Wc, ba, bb, bc, w, gamma, beta):
    return pl.pallas_call(
        _tc_layer1_body,
        out_shape=jax.ShapeDtypeStruct((NPAD, D), jnp.float32),
    )(p, norms, Wa, Wb, Wc, ba.reshape(1, D), bb.reshape(1, D),
      bc.reshape(1, D), w.reshape(1, 3), gamma.reshape(1, D),
      beta.reshape(1, D))


def _tc_layer2_body(p_ref, norms_ref, wa_ref, wb_ref, wc_ref,
                    ba_ref, bb_ref, bc_ref, w_ref, out_ref):
    w = w_ref[...]
    wn = w / jnp.sum(w)
    W = (wn[:, 0:1] * wa_ref[...] + wn[:, 1:2] * wb_ref[...]
         + wn[:, 2:3] * wc_ref[...])
    b = (wn[:, 0:1] * ba_ref[...] + wn[:, 1:2] * bb_ref[...]
         + wn[:, 2:3] * bc_ref[...])
    agg = p_ref[0] + p_ref[1]
    a = agg * norms_ref[:, 1:2]
    o = lax.dot_general(a, W, (((1,), (0,)), ((), ())),
                        preferred_element_type=jnp.float32) + b
    m = jnp.max(o, axis=1, keepdims=True)
    ex = jnp.exp(o - m)
    out_ref[...] = (o - m) - jnp.log(jnp.sum(ex, axis=1, keepdims=True))


def _tc_layer2(p, norms, Wa, Wb, Wc, ba, bb, bc, w):
    return pl.pallas_call(
        _tc_layer2_body,
        out_shape=jax.ShapeDtypeStruct((NPAD, DOUT), jnp.float32),
    )(p, norms, Wa, Wb, Wc, ba.reshape(1, DOUT), bb.reshape(1, DOUT),
      bc.reshape(1, DOUT), w.reshape(1, 3))


def kernel(x, edge_index, W1a, b1a, W1b, b1b, W1c, b1c, w1, bn_gamma,
           bn_beta, W2a, b2a, W2b, b2b, W2c, b2c, w2):
    src = edge_index[0]
    dst = edge_index[1]
    src3 = src.reshape(NW, NCHUNK, K)
    dst3 = dst.reshape(NW, NCHUNK, K)
    deg_p = _make_sc_degrees()(src, dst)
    hs, norms = _tc_prep(deg_p, x)
    p1 = _make_sc_agg(N)(hs, src3, dst3)
    hs2 = _tc_layer1(p1, norms, W1a, W1b, W1c, b1a, b1b, b1c, w1,
                     bn_gamma, bn_beta)
    p2 = _make_sc_agg(NPAD)(hs2, src3, dst3)
    out = _tc_layer2(p2, norms, W2a, W2b, W2c, b2a, b2b, b2c, w2)
    return out[:N]
